# Initial kernel scaffold; baseline (speedup 1.0000x reference)
import functools

import jax
import jax.numpy as jnp
from jax import lax
from jax.experimental import pallas as pl
from jax.experimental.pallas import tpu as pltpu

D_MODEL = 1024
D_FF = 4096
N_EXPERTS = 16
TOP_K = 2
AUX_COEF = 0.01

BLK_M = 256
N_TOKENS = 2 * 4096
N_PAIRS = N_TOKENS * TOP_K
R_PAD = N_PAIRS + N_EXPERTS * BLK_M
NB = R_PAD // BLK_M


def _ffn_body(eob_ref, xg_ref, w1_ref, b1_ref, w2_ref, b2_ref, wr_ref,
              out_ref):
    x = xg_ref[...]
    h = jnp.dot(x, w1_ref[0], preferred_element_type=jnp.float32)
    h = h + b1_ref[...]
    h = h * jax.nn.sigmoid(h)
    o = jnp.dot(h.astype(jnp.bfloat16), w2_ref[0],
                preferred_element_type=jnp.float32)
    out_ref[...] = ((o + b2_ref[...]) * wr_ref[...]).astype(jnp.bfloat16)


def _expert_ffn(eob, xg, w1, b1, w2, b2, w_row):
    grid_spec = pltpu.PrefetchScalarGridSpec(
        num_scalar_prefetch=1,
        grid=(NB,),
        in_specs=[
            pl.BlockSpec((BLK_M, D_MODEL), lambda i, eob: (i, 0)),
            pl.BlockSpec((1, D_MODEL, D_FF), lambda i, eob: (eob[i], 0, 0)),
            pl.BlockSpec((1, D_FF), lambda i, eob: (eob[i], 0)),
            pl.BlockSpec((1, D_FF, D_MODEL), lambda i, eob: (eob[i], 0, 0)),
            pl.BlockSpec((1, D_MODEL), lambda i, eob: (eob[i], 0)),
            pl.BlockSpec((BLK_M, 1), lambda i, eob: (i, 0)),
        ],
        out_specs=pl.BlockSpec((BLK_M, D_MODEL), lambda i, eob: (i, 0)),
    )
    return pl.pallas_call(
        _ffn_body,
        grid_spec=grid_spec,
        out_shape=jax.ShapeDtypeStruct((R_PAD, D_MODEL), jnp.bfloat16),
    )(eob, xg, w1, b1, w2, b2, w_row)


def kernel(x, Wg, W1, b1, W2, b2):
    B, T, D = x.shape
    x_flat = x.reshape(N_TOKENS, D)

    logits = x_flat @ Wg
    probs = jax.nn.softmax(logits, axis=-1)
    top_w, top_i = lax.top_k(probs, TOP_K)
    top_w = top_w / (top_w.sum(axis=-1, keepdims=True) + 1e-9)

    e_flat = top_i.reshape(-1).astype(jnp.int32)
    oh = (e_flat[:, None] == jnp.arange(N_EXPERTS, dtype=jnp.int32)[None, :])
    ohi = oh.astype(jnp.int32)
    within = jnp.cumsum(ohi, axis=0)
    rank = (within * ohi).sum(axis=1) - 1
    counts = ohi.sum(axis=0)
    padded = ((counts + BLK_M - 1) // BLK_M) * BLK_M
    starts = jnp.concatenate(
        [jnp.zeros((1,), jnp.int32), jnp.cumsum(padded)[:-1]])
    rows = starts[e_flat] + rank

    block_starts = starts // BLK_M
    bidx = jnp.arange(NB, dtype=jnp.int32)
    eob = (bidx[:, None] >= block_starts[None, :]).sum(axis=1) - 1
    eob = eob.astype(jnp.int32)

    tok_of_row = jnp.zeros((R_PAD,), jnp.int32).at[rows].set(
        jnp.arange(N_PAIRS, dtype=jnp.int32) // TOP_K)
    w_of_row = jnp.zeros((R_PAD,), jnp.float32).at[rows].set(
        top_w.reshape(-1))

    x_bf = x_flat.astype(jnp.bfloat16)
    xg = x_bf[tok_of_row]

    out_g = _expert_ffn(eob, xg,
                        W1.astype(jnp.bfloat16), b1,
                        W2.astype(jnp.bfloat16), b2,
                        w_of_row[:, None])

    gathered = out_g[rows].astype(jnp.float32)
    out = gathered.reshape(N_TOKENS, TOP_K, D).sum(axis=1)
    out = out.reshape(B, T, D)

    top1 = top_i[:, 0]
    f = (jax.nn.one_hot(top1, N_EXPERTS, dtype=jnp.float32)).mean(axis=0)
    P = probs.mean(axis=0)
    aux_loss = AUX_COEF * N_EXPERTS * (f * P).sum()
    return (out, aux_loss)


# speedup vs baseline: 1.9276x; 1.9276x over previous
import functools

import jax
import jax.numpy as jnp
from jax import lax
from jax.experimental import pallas as pl
from jax.experimental.pallas import tpu as pltpu

D_MODEL = 1024
D_FF = 4096
N_EXPERTS = 16
TOP_K = 2
AUX_COEF = 0.01

BLK_M = 256
N_TOKENS = 2 * 4096
N_PAIRS = N_TOKENS * TOP_K
R_PAD = N_PAIRS + N_EXPERTS * BLK_M
NB = R_PAD // BLK_M


def _ffn_body(eob_ref, xg_ref, w1_ref, b1_ref, w2_ref, b2_ref, wr_ref,
              out_ref):
    x = xg_ref[...]
    h = jnp.dot(x, w1_ref[0], preferred_element_type=jnp.float32)
    h = h + b1_ref[0]
    h = h * jax.nn.sigmoid(h)
    o = jnp.dot(h.astype(jnp.bfloat16), w2_ref[0],
                preferred_element_type=jnp.float32)
    out_ref[...] = ((o + b2_ref[0]) * wr_ref[...]).astype(jnp.bfloat16)


def _expert_ffn(eob, xg, w1, b1, w2, b2, w_row):
    grid_spec = pltpu.PrefetchScalarGridSpec(
        num_scalar_prefetch=1,
        grid=(NB,),
        in_specs=[
            pl.BlockSpec((BLK_M, D_MODEL), lambda i, eob: (i, 0)),
            pl.BlockSpec((1, D_MODEL, D_FF), lambda i, eob: (eob[i], 0, 0)),
            pl.BlockSpec((1, 1, D_FF), lambda i, eob: (eob[i], 0, 0)),
            pl.BlockSpec((1, D_FF, D_MODEL), lambda i, eob: (eob[i], 0, 0)),
            pl.BlockSpec((1, 1, D_MODEL), lambda i, eob: (eob[i], 0, 0)),
            pl.BlockSpec((BLK_M, 1), lambda i, eob: (i, 0)),
        ],
        out_specs=pl.BlockSpec((BLK_M, D_MODEL), lambda i, eob: (i, 0)),
    )
    return pl.pallas_call(
        _ffn_body,
        grid_spec=grid_spec,
        out_shape=jax.ShapeDtypeStruct((R_PAD, D_MODEL), jnp.bfloat16),
    )(eob, xg, w1, b1, w2, b2, w_row)


def kernel(x, Wg, W1, b1, W2, b2):
    B, T, D = x.shape
    x_flat = x.reshape(N_TOKENS, D)

    logits = x_flat @ Wg
    probs = jax.nn.softmax(logits, axis=-1)
    top_w, top_i = lax.top_k(probs, TOP_K)
    top_w = top_w / (top_w.sum(axis=-1, keepdims=True) + 1e-9)

    e_flat = top_i.reshape(-1).astype(jnp.int32)
    oh = (e_flat[:, None] == jnp.arange(N_EXPERTS, dtype=jnp.int32)[None, :])
    ohi = oh.astype(jnp.int32)
    within = jnp.cumsum(ohi, axis=0)
    rank = (within * ohi).sum(axis=1) - 1
    counts = ohi.sum(axis=0)
    padded = ((counts + BLK_M - 1) // BLK_M) * BLK_M
    starts = jnp.concatenate(
        [jnp.zeros((1,), jnp.int32), jnp.cumsum(padded)[:-1]])
    rows = starts[e_flat] + rank

    block_starts = starts // BLK_M
    bidx = jnp.arange(NB, dtype=jnp.int32)
    eob = (bidx[:, None] >= block_starts[None, :]).sum(axis=1) - 1
    eob = eob.astype(jnp.int32)

    tok_of_row = jnp.zeros((R_PAD,), jnp.int32).at[rows].set(
        jnp.arange(N_PAIRS, dtype=jnp.int32) // TOP_K)
    w_of_row = jnp.zeros((R_PAD,), jnp.float32).at[rows].set(
        top_w.reshape(-1))

    x_bf = x_flat.astype(jnp.bfloat16)
    xg = x_bf[tok_of_row]

    out_g = _expert_ffn(eob, xg,
                        W1.astype(jnp.bfloat16), b1[:, None, :],
                        W2.astype(jnp.bfloat16), b2[:, None, :],
                        w_of_row[:, None])

    gathered = out_g[rows].astype(jnp.float32)
    out = gathered.reshape(N_TOKENS, TOP_K, D).sum(axis=1)
    out = out.reshape(B, T, D)

    top1 = top_i[:, 0]
    f = (jax.nn.one_hot(top1, N_EXPERTS, dtype=jnp.float32)).mean(axis=0)
    P = probs.mean(axis=0)
    aux_loss = AUX_COEF * N_EXPERTS * (f * P).sum()
    return (out, aux_loss)


# speedup vs baseline: 1.9984x; 1.0367x over previous
import functools

import jax
import jax.numpy as jnp
from jax import lax
from jax.experimental import pallas as pl
from jax.experimental.pallas import tpu as pltpu
from jax.experimental.pallas import tpu_sc as plsc

D_MODEL = 1024
D_FF = 4096
N_EXPERTS = 16
TOP_K = 2
AUX_COEF = 0.01

N_TOKENS = 2 * 4096
N_PAIRS = N_TOKENS * TOP_K
BLK_M = 256
R_PAD = N_PAIRS + N_EXPERTS * BLK_M
NB = R_PAD // BLK_M

_SC_MESH = dict(core_axis_name="c", subcore_axis_name="s")
NC, NS = 2, 16
NW = NC * NS
L = 16


_RT_BLK = 512
_RT_GRID = N_TOKENS // _RT_BLK


def _router_body(x_ref, wg_ref, i1_ref, i2_ref, w1_ref, w2_ref,
                 ps_ref, cs_ref, aux_ref):
    i = pl.program_id(0)
    logits = jnp.dot(x_ref[...], wg_ref[...],
                     preferred_element_type=jnp.float32)
    m = jnp.max(logits, axis=1, keepdims=True)
    ex = jnp.exp(logits - m)
    probs = ex / jnp.sum(ex, axis=1, keepdims=True)

    idx16 = lax.broadcasted_iota(jnp.int32, (_RT_BLK, N_EXPERTS), 1)
    p1 = jnp.max(probs, axis=1)
    i1 = jnp.min(jnp.where(probs == p1[:, None], idx16, N_EXPERTS), axis=1)
    masked = jnp.where(idx16 == i1[:, None], -1.0, probs)
    p2 = jnp.max(masked, axis=1)
    i2 = jnp.min(jnp.where(masked == p2[:, None], idx16, N_EXPERTS), axis=1)
    denom = p1 + p2 + 1e-9
    i1_ref[...] = i1
    i2_ref[...] = i2
    w1_ref[...] = p1 / denom
    w2_ref[...] = p2 / denom

    @pl.when(i == 0)
    def _():
        ps_ref[...] = jnp.zeros((N_EXPERTS,), jnp.float32)
        cs_ref[...] = jnp.zeros((N_EXPERTS,), jnp.float32)

    ps_ref[...] += jnp.sum(probs, axis=0)
    cs_ref[...] += jnp.sum((idx16 == i1[:, None]).astype(jnp.float32), axis=0)

    @pl.when(i == _RT_GRID - 1)
    def _():
        f = cs_ref[...] / N_TOKENS
        P = ps_ref[...] / N_TOKENS
        aux_ref[...] = jnp.full((1, 1), AUX_COEF * N_EXPERTS * jnp.sum(f * P),
                                jnp.float32)


def _router(x_flat, Wg):
    return pl.pallas_call(
        _router_body,
        grid=(_RT_GRID,),
        in_specs=[
            pl.BlockSpec((_RT_BLK, D_MODEL), lambda i: (i, 0)),
            pl.BlockSpec((D_MODEL, N_EXPERTS), lambda i: (0, 0)),
        ],
        out_specs=[
            pl.BlockSpec((_RT_BLK,), lambda i: (i,)),
            pl.BlockSpec((_RT_BLK,), lambda i: (i,)),
            pl.BlockSpec((_RT_BLK,), lambda i: (i,)),
            pl.BlockSpec((_RT_BLK,), lambda i: (i,)),
            pl.BlockSpec((N_EXPERTS,), lambda i: (0,)),
            pl.BlockSpec((N_EXPERTS,), lambda i: (0,)),
            pl.BlockSpec((1, 1), lambda i: (0, 0)),
        ],
        out_shape=[
            jax.ShapeDtypeStruct((N_TOKENS,), jnp.int32),
            jax.ShapeDtypeStruct((N_TOKENS,), jnp.int32),
            jax.ShapeDtypeStruct((N_TOKENS,), jnp.float32),
            jax.ShapeDtypeStruct((N_TOKENS,), jnp.float32),
            jax.ShapeDtypeStruct((N_EXPERTS,), jnp.float32),
            jax.ShapeDtypeStruct((N_EXPERTS,), jnp.float32),
            jax.ShapeDtypeStruct((1, 1), jnp.float32),
        ],
    )(x_flat, Wg)


_DP_CHUNK = N_PAIRS // NS
_DP_SUB = 512


def _dispatch_body(e_hbm, rows_hbm, eob_hbm,
                   ids_v, cnt16_v, cnts_v, dest_v, eob_v,
                   cnt_s, base_s, bstart_s,
                   counts_sh, sem):
    cid = lax.axis_index("c")
    sid = lax.axis_index("s")

    @pl.when(cid == 0)
    def _():
        lanes = lax.iota(jnp.int32, L)
        base_off = sid * _DP_CHUNK
        pltpu.sync_copy(e_hbm.at[pl.ds(base_off, _DP_CHUNK)],
                        ids_v.at[pl.ds(0, _DP_CHUNK)])

        def zero_body(i, _):
            cnt_s[i] = 0
            return 0
        lax.fori_loop(0, N_EXPERTS, zero_body, 0)

        def hist_body(i, _):
            e = ids_v[pl.ds(i, L)][0]
            cnt_s[e] = cnt_s[e] + 1
            return 0
        lax.fori_loop(0, _DP_CHUNK, hist_body, 0)

        cvec = jnp.zeros((L,), jnp.int32)
        for e in range(N_EXPERTS):
            cvec = jnp.where(lanes == e, cnt_s[e], cvec)
        cnt16_v[...] = cvec
        pltpu.sync_copy(cnt16_v,
                        counts_sh.at[pl.ds(sid * N_EXPERTS, N_EXPERTS)])
        plsc.subcore_barrier()

        pltpu.sync_copy(counts_sh, cnts_v.at[pl.ds(0, NS * N_EXPERTS)])

        def _cnt(w, e):
            return cnts_v[pl.ds(w * N_EXPERTS + e, L)][0]

        def prior_body(w, acc):
            take = w < sid
            return tuple(
                acc[e] + jnp.where(take, _cnt(w, e), 0)
                for e in range(N_EXPERTS))
        prior = lax.fori_loop(0, NS, prior_body,
                              tuple(jnp.int32(0) for _ in range(N_EXPERTS)))

        start = jnp.int32(0)
        for e in range(N_EXPERTS):
            tot = _cnt(0, e)
            for w in range(1, NS):
                tot = tot + _cnt(w, e)
            base_s[e] = start + prior[e]
            bstart_s[e] = start // BLK_M
            start = start + ((tot + (BLK_M - 1)) // BLK_M) * BLK_M

        @pl.when(sid == 0)
        def _():
            for j in range(NB // L):
                bv = lanes + j * L
                c = jnp.full((L,), -1, jnp.int32)
                for e in range(N_EXPERTS):
                    c = c + jnp.where(bv >= bstart_s[e], 1, 0)
                eob_v[pl.ds(j * L, L)] = c
            pltpu.sync_copy(eob_v, eob_hbm)

        def rank_body(v, _):
            dvec = jnp.zeros((L,), jnp.int32)
            for j in range(L):
                e = ids_v[pl.ds(v * L + j, L)][0]
                r = base_s[e]
                base_s[e] = r + 1
                dvec = jnp.where(lanes == j, r, dvec)
            dest_v[pl.ds(v * L, L)] = dvec
            return 0
        lax.fori_loop(0, _DP_CHUNK // L, rank_body, 0)
        pltpu.sync_copy(dest_v, rows_hbm.at[pl.ds(base_off, _DP_CHUNK)])


def _dispatch(e_flat):
    mesh = plsc.VectorSubcoreMesh(**_SC_MESH)
    f = pl.kernel(
        _dispatch_body,
        out_type=[
            jax.ShapeDtypeStruct((N_PAIRS,), jnp.int32),
            jax.ShapeDtypeStruct((NB,), jnp.int32),
        ],
        mesh=mesh,
        scratch_types=[
            pltpu.VMEM((_DP_CHUNK + L,), jnp.int32),
            pltpu.VMEM((L,), jnp.int32),
            pltpu.VMEM((NS * N_EXPERTS + L,), jnp.int32),
            pltpu.VMEM((_DP_CHUNK,), jnp.int32),
            pltpu.VMEM((NB,), jnp.int32),
            pltpu.SMEM((N_EXPERTS,), jnp.int32),
            pltpu.SMEM((N_EXPERTS,), jnp.int32),
            pltpu.SMEM((N_EXPERTS,), jnp.int32),
            pltpu.VMEM_SHARED((NS * N_EXPERTS,), jnp.int32),
            pltpu.SemaphoreType.DMA,
        ],
    )
    return f(e_flat)


_GW = D_MODEL // 2
_G_PER_W = N_PAIRS // NW
_G_CH = 64
_G_NCH = _G_PER_W // _G_CH


def _dgather_body(x_hbm, rows_hbm, xg_hbm, didx_v, tok_v, buf_v, sem):
    wid = lax.axis_index("s") * NC + lax.axis_index("c")
    base = wid * _G_PER_W
    for j in range(_G_NCH):
        pltpu.sync_copy(rows_hbm.at[pl.ds(base + j * _G_CH, _G_CH)],
                        didx_v.at[j])
    lanes = lax.iota(jnp.int32, L)
    for v in range(_G_PER_W // L):
        pv = lanes + (base + v * L)
        tv = jnp.where(pv >= N_TOKENS, pv - N_TOKENS, pv)
        tok_v[pl.ds(v * L, L)] = tv
    for j in range(_G_NCH):
        pltpu.async_copy(x_hbm.at[tok_v.at[pl.ds(j * _G_CH, _G_CH)]],
                         buf_v, sem).wait()
        pltpu.async_copy(buf_v, xg_hbm.at[didx_v.at[j]], sem).wait()


def _dgather(x_i32, rows):
    mesh = plsc.VectorSubcoreMesh(**_SC_MESH)
    f = pl.kernel(
        _dgather_body,
        out_type=jax.ShapeDtypeStruct((R_PAD, _GW), jnp.int32),
        mesh=mesh,
        scratch_types=[
            pltpu.VMEM((_G_NCH, _G_CH), jnp.int32),
            pltpu.VMEM((_G_PER_W,), jnp.int32),
            pltpu.VMEM((_G_CH, _GW), jnp.int32),
            pltpu.SemaphoreType.DMA,
        ],
    )
    return f(x_i32, rows)


def _ffn_body(eob_ref, xg_ref, w1_ref, b1_ref, w2_ref, b2_ref, out_ref):
    x = xg_ref[...]
    h = jnp.dot(x, w1_ref[0], preferred_element_type=jnp.float32)
    h = h + b1_ref[0]
    h = h * jax.nn.sigmoid(h)
    o = jnp.dot(h.astype(jnp.bfloat16), w2_ref[0],
                preferred_element_type=jnp.float32)
    out_ref[...] = o + b2_ref[0]


def _expert_ffn(eob, xg, w1, b1, w2, b2):
    grid_spec = pltpu.PrefetchScalarGridSpec(
        num_scalar_prefetch=1,
        grid=(NB,),
        in_specs=[
            pl.BlockSpec((BLK_M, D_MODEL), lambda i, eob: (i, 0)),
            pl.BlockSpec((1, D_MODEL, D_FF), lambda i, eob: (eob[i], 0, 0)),
            pl.BlockSpec((1, 1, D_FF), lambda i, eob: (eob[i], 0, 0)),
            pl.BlockSpec((1, D_FF, D_MODEL), lambda i, eob: (eob[i], 0, 0)),
            pl.BlockSpec((1, 1, D_MODEL), lambda i, eob: (eob[i], 0, 0)),
        ],
        out_specs=pl.BlockSpec((BLK_M, D_MODEL), lambda i, eob: (i, 0)),
    )
    return pl.pallas_call(
        _ffn_body,
        grid_spec=grid_spec,
        out_shape=jax.ShapeDtypeStruct((R_PAD, D_MODEL), jnp.float32),
    )(eob, xg, w1, b1, w2, b2)


_C_PER_W = N_TOKENS // NW
_C_CH = 32
_C_COLS = D_MODEL // L


def _combine_body(outg_hbm, rows_hbm, w_hbm, out_hbm,
                  i0_v, i1_v, w0_v, w1_v, g0_v, g1_v, ob_v, sem0, sem1):
    wid = lax.axis_index("s") * NC + lax.axis_index("c")
    base = wid * _C_PER_W
    def body(ci, _):
        off = base + ci * _C_CH
        pltpu.sync_copy(rows_hbm.at[pl.ds(off, _C_CH)], i0_v)
        pltpu.sync_copy(rows_hbm.at[pl.ds(N_TOKENS + off, _C_CH)], i1_v)
        pltpu.sync_copy(w_hbm.at[pl.ds(off, _C_CH)],
                        w0_v.at[pl.ds(0, _C_CH)])
        pltpu.sync_copy(w_hbm.at[pl.ds(N_TOKENS + off, _C_CH)],
                        w1_v.at[pl.ds(0, _C_CH)])
        cp0 = pltpu.async_copy(outg_hbm.at[i0_v], g0_v, sem0)
        cp1 = pltpu.async_copy(outg_hbm.at[i1_v], g1_v, sem1)
        cp0.wait()
        cp1.wait()
        def tok_body(r, _):
            w0 = w0_v[pl.ds(r, L)][0]
            w1 = w1_v[pl.ds(r, L)][0]
            for c in range(_C_COLS):
                ob_v[r, pl.ds(c * L, L)] = (
                    w0 * g0_v[r, pl.ds(c * L, L)]
                    + w1 * g1_v[r, pl.ds(c * L, L)])
            return 0
        lax.fori_loop(0, _C_CH, tok_body, 0)
        pltpu.sync_copy(ob_v, out_hbm.at[pl.ds(off, _C_CH)])
        return 0
    lax.fori_loop(0, _C_PER_W // _C_CH, body, 0)


def _combine(out_g, rows, w_pairs):
    mesh = plsc.VectorSubcoreMesh(**_SC_MESH)
    f = pl.kernel(
        _combine_body,
        out_type=jax.ShapeDtypeStruct((N_TOKENS, D_MODEL), jnp.float32),
        mesh=mesh,
        scratch_types=[
            pltpu.VMEM((_C_CH,), jnp.int32),
            pltpu.VMEM((_C_CH,), jnp.int32),
            pltpu.VMEM((_C_CH + L,), jnp.float32),
            pltpu.VMEM((_C_CH + L,), jnp.float32),
            pltpu.VMEM((_C_CH, D_MODEL), jnp.float32),
            pltpu.VMEM((_C_CH, D_MODEL), jnp.float32),
            pltpu.VMEM((_C_CH, D_MODEL), jnp.float32),
            pltpu.SemaphoreType.DMA,
            pltpu.SemaphoreType.DMA,
        ],
    )
    return f(out_g, rows, w_pairs)


def kernel(x, Wg, W1, b1, W2, b2):
    B, T, D = x.shape
    x_flat = x.reshape(N_TOKENS, D)

    i1, i2, w1n, w2n, _ps, _cs, aux = _router(x_flat, Wg)
    e_flat = jnp.concatenate([i1, i2])
    w_pairs = jnp.concatenate([w1n, w2n])

    rows, eob = _dispatch(e_flat)

    x_bf = x_flat.astype(jnp.bfloat16)
    x_i32 = lax.bitcast_convert_type(
        x_bf.reshape(N_TOKENS, _GW, 2), jnp.int32)
    xg_i32 = _dgather(x_i32, rows)
    xg = lax.bitcast_convert_type(xg_i32, jnp.bfloat16).reshape(R_PAD, D)

    out_g = _expert_ffn(eob, xg,
                        W1.astype(jnp.bfloat16), b1[:, None, :],
                        W2.astype(jnp.bfloat16), b2[:, None, :])

    out = _combine(out_g, rows, w_pairs).reshape(B, T, D)
    return (out, aux.reshape(()))


# speedup vs baseline: 3.6978x; 1.8504x over previous
import functools

import jax
import jax.numpy as jnp
from jax import lax
from jax.experimental import pallas as pl
from jax.experimental.pallas import tpu as pltpu
from jax.experimental.pallas import tpu_sc as plsc

D_MODEL = 1024
D_FF = 4096
N_EXPERTS = 16
TOP_K = 2
AUX_COEF = 0.01

N_TOKENS = 2 * 4096
N_PAIRS = N_TOKENS * TOP_K
BLK_M = 256
R_PAD = N_PAIRS + N_EXPERTS * BLK_M
NB = R_PAD // BLK_M

_SC_MESH = dict(core_axis_name="c", subcore_axis_name="s")
NC, NS = 2, 16
NW = NC * NS
L = 16


_RT_BLK = 512
_RT_GRID = N_TOKENS // _RT_BLK


def _router_body(x_ref, wg_ref, e_ref, w_ref,
                 ps_ref, cs_ref, aux_ref):
    i = pl.program_id(0)
    logits = jnp.dot(x_ref[...], wg_ref[...],
                     preferred_element_type=jnp.float32)
    m = jnp.max(logits, axis=1, keepdims=True)
    ex = jnp.exp(logits - m)
    probs = ex / jnp.sum(ex, axis=1, keepdims=True)

    idx16 = lax.broadcasted_iota(jnp.int32, (_RT_BLK, N_EXPERTS), 1)
    p1 = jnp.max(probs, axis=1)
    i1 = jnp.min(jnp.where(probs == p1[:, None], idx16, N_EXPERTS), axis=1)
    masked = jnp.where(idx16 == i1[:, None], -1.0, probs)
    p2 = jnp.max(masked, axis=1)
    i2 = jnp.min(jnp.where(masked == p2[:, None], idx16, N_EXPERTS), axis=1)
    denom = p1 + p2 + 1e-9
    e_ref[...] = jnp.concatenate([i1[None, :], i2[None, :]], axis=0)
    w_ref[...] = jnp.concatenate([(p1 / denom)[None, :],
                                  (p2 / denom)[None, :]], axis=0)

    @pl.when(i == 0)
    def _():
        ps_ref[...] = jnp.zeros((N_EXPERTS,), jnp.float32)
        cs_ref[...] = jnp.zeros((N_EXPERTS,), jnp.float32)

    ps_ref[...] += jnp.sum(probs, axis=0)
    cs_ref[...] += jnp.sum((idx16 == i1[:, None]).astype(jnp.float32), axis=0)

    @pl.when(i == _RT_GRID - 1)
    def _():
        f = cs_ref[...] / N_TOKENS
        P = ps_ref[...] / N_TOKENS
        aux_ref[...] = jnp.full((1, 1), AUX_COEF * N_EXPERTS * jnp.sum(f * P),
                                jnp.float32)


def _router(x_flat, Wg):
    return pl.pallas_call(
        _router_body,
        grid=(_RT_GRID,),
        in_specs=[
            pl.BlockSpec((_RT_BLK, D_MODEL), lambda i: (i, 0)),
            pl.BlockSpec((D_MODEL, N_EXPERTS), lambda i: (0, 0)),
        ],
        out_specs=[
            pl.BlockSpec((TOP_K, _RT_BLK), lambda i: (0, i)),
            pl.BlockSpec((TOP_K, _RT_BLK), lambda i: (0, i)),
            pl.BlockSpec((N_EXPERTS,), lambda i: (0,)),
            pl.BlockSpec((N_EXPERTS,), lambda i: (0,)),
            pl.BlockSpec((1, 1), lambda i: (0, 0)),
        ],
        out_shape=[
            jax.ShapeDtypeStruct((TOP_K, N_TOKENS), jnp.int32),
            jax.ShapeDtypeStruct((TOP_K, N_TOKENS), jnp.float32),
            jax.ShapeDtypeStruct((N_EXPERTS,), jnp.float32),
            jax.ShapeDtypeStruct((N_EXPERTS,), jnp.float32),
            jax.ShapeDtypeStruct((1, 1), jnp.float32),
        ],
    )(x_flat, Wg)


_DP_CHUNK = N_PAIRS // NS
_DP_SUB = 512


def _dispatch_body(e_hbm, rows_hbm, eob_hbm,
                   ids_v, cnt16_v, cnts_v, dest_v, eob_v,
                   cnt_s, base_s, bstart_s,
                   counts_sh, sem):
    cid = lax.axis_index("c")
    sid = lax.axis_index("s")

    @pl.when(cid == 0)
    def _():
        lanes = lax.iota(jnp.int32, L)
        base_off = sid * _DP_CHUNK
        pltpu.sync_copy(e_hbm.at[pl.ds(base_off, _DP_CHUNK)],
                        ids_v.at[pl.ds(0, _DP_CHUNK)])

        def zero_body(i, _):
            cnt_s[i] = 0
            return 0
        lax.fori_loop(0, N_EXPERTS, zero_body, 0)

        def hist_body(i, _):
            e = ids_v[pl.ds(i, L)][0]
            cnt_s[e] = cnt_s[e] + 1
            return 0
        lax.fori_loop(0, _DP_CHUNK, hist_body, 0)

        cvec = jnp.zeros((L,), jnp.int32)
        for e in range(N_EXPERTS):
            cvec = jnp.where(lanes == e, cnt_s[e], cvec)
        cnt16_v[...] = cvec
        pltpu.sync_copy(cnt16_v,
                        counts_sh.at[pl.ds(sid * N_EXPERTS, N_EXPERTS)])
        plsc.subcore_barrier()

        pltpu.sync_copy(counts_sh, cnts_v.at[pl.ds(0, NS * N_EXPERTS)])

        def _cnt(w, e):
            return cnts_v[pl.ds(w * N_EXPERTS + e, L)][0]

        def prior_body(w, acc):
            take = w < sid
            return tuple(
                acc[e] + jnp.where(take, _cnt(w, e), 0)
                for e in range(N_EXPERTS))
        prior = lax.fori_loop(0, NS, prior_body,
                              tuple(jnp.int32(0) for _ in range(N_EXPERTS)))

        start = jnp.int32(0)
        for e in range(N_EXPERTS):
            tot = _cnt(0, e)
            for w in range(1, NS):
                tot = tot + _cnt(w, e)
            base_s[e] = start + prior[e]
            bstart_s[e] = start // BLK_M
            start = start + ((tot + (BLK_M - 1)) // BLK_M) * BLK_M

        @pl.when(sid == 0)
        def _():
            for j in range(NB // L):
                bv = lanes + j * L
                c = jnp.full((L,), -1, jnp.int32)
                for e in range(N_EXPERTS):
                    c = c + jnp.where(bv >= bstart_s[e], 1, 0)
                eob_v[pl.ds(j * L, L)] = c
            pltpu.sync_copy(eob_v, eob_hbm)

        def rank_body(v, _):
            dvec = jnp.zeros((L,), jnp.int32)
            for j in range(L):
                e = ids_v[pl.ds(v * L + j, L)][0]
                r = base_s[e]
                base_s[e] = r + 1
                dvec = jnp.where(lanes == j, r, dvec)
            dest_v[pl.ds(v * L, L)] = dvec
            return 0
        lax.fori_loop(0, _DP_CHUNK // L, rank_body, 0)
        pltpu.sync_copy(dest_v, rows_hbm.at[pl.ds(base_off, _DP_CHUNK)])


def _dispatch(e_flat):
    mesh = plsc.VectorSubcoreMesh(**_SC_MESH)
    f = pl.kernel(
        _dispatch_body,
        out_type=[
            jax.ShapeDtypeStruct((N_PAIRS,), jnp.int32),
            jax.ShapeDtypeStruct((NB,), jnp.int32),
        ],
        mesh=mesh,
        scratch_types=[
            pltpu.VMEM((_DP_CHUNK + L,), jnp.int32),
            pltpu.VMEM((L,), jnp.int32),
            pltpu.VMEM((NS * N_EXPERTS + L,), jnp.int32),
            pltpu.VMEM((_DP_CHUNK,), jnp.int32),
            pltpu.VMEM((NB,), jnp.int32),
            pltpu.SMEM((N_EXPERTS,), jnp.int32),
            pltpu.SMEM((N_EXPERTS,), jnp.int32),
            pltpu.SMEM((N_EXPERTS,), jnp.int32),
            pltpu.VMEM_SHARED((NS * N_EXPERTS,), jnp.int32),
            pltpu.SemaphoreType.DMA,
        ],
    )
    return f(e_flat)


_GW = D_MODEL
_G_PER_W = N_PAIRS // NW
_G_CH = 64
_G_NCH = _G_PER_W // _G_CH


def _dgather_body(x_hbm, rows_hbm, xg_hbm, didx_v, tok_v, buf_v, sem):
    wid = lax.axis_index("s") * NC + lax.axis_index("c")
    base = wid * _G_PER_W
    for j in range(_G_NCH):
        pltpu.sync_copy(rows_hbm.at[pl.ds(base + j * _G_CH, _G_CH)],
                        didx_v.at[j])
    lanes = lax.iota(jnp.int32, L)
    for v in range(_G_PER_W // L):
        pv = lanes + (base + v * L)
        tv = jnp.where(pv >= N_TOKENS, pv - N_TOKENS, pv)
        tok_v[pl.ds(v * L, L)] = tv
    for j in range(_G_NCH):
        pltpu.async_copy(x_hbm.at[tok_v.at[pl.ds(j * _G_CH, _G_CH)]],
                         buf_v, sem).wait()
        pltpu.async_copy(buf_v, xg_hbm.at[didx_v.at[j]], sem).wait()


def _dgather(x_i32, rows):
    mesh = plsc.VectorSubcoreMesh(**_SC_MESH)
    f = pl.kernel(
        _dgather_body,
        out_type=jax.ShapeDtypeStruct((R_PAD, _GW), jnp.float32),
        mesh=mesh,
        scratch_types=[
            pltpu.VMEM((_G_NCH, _G_CH), jnp.int32),
            pltpu.VMEM((_G_PER_W,), jnp.int32),
            pltpu.VMEM((_G_CH, _GW), jnp.float32),
            pltpu.SemaphoreType.DMA,
        ],
    )
    return f(x_i32, rows)


def _ffn1_body(eob_ref, xg_ref, w1_ref, b1_ref, h_ref):
    h = jnp.dot(xg_ref[...], w1_ref[0], preferred_element_type=jnp.float32)
    h = h + b1_ref[0]
    h_ref[...] = (h * jax.nn.sigmoid(h)).astype(jnp.bfloat16)


def _ffn2_body(eob_ref, h_ref, w2_ref, b2_ref, out_ref):
    o = jnp.dot(h_ref[...].astype(jnp.float32), w2_ref[0],
                preferred_element_type=jnp.float32)
    out_ref[...] = o + b2_ref[0]


def _expert_ffn(eob, xg, w1, b1, w2, b2):
    gs1 = pltpu.PrefetchScalarGridSpec(
        num_scalar_prefetch=1,
        grid=(NB,),
        in_specs=[
            pl.BlockSpec((BLK_M, D_MODEL), lambda i, eob: (i, 0)),
            pl.BlockSpec((1, D_MODEL, D_FF), lambda i, eob: (eob[i], 0, 0)),
            pl.BlockSpec((1, 1, D_FF), lambda i, eob: (eob[i], 0, 0)),
        ],
        out_specs=pl.BlockSpec((BLK_M, D_FF), lambda i, eob: (i, 0)),
    )
    h = pl.pallas_call(
        _ffn1_body,
        grid_spec=gs1,
        out_shape=jax.ShapeDtypeStruct((R_PAD, D_FF), jnp.bfloat16),
    )(eob, xg, w1, b1)
    gs2 = pltpu.PrefetchScalarGridSpec(
        num_scalar_prefetch=1,
        grid=(NB,),
        in_specs=[
            pl.BlockSpec((BLK_M, D_FF), lambda i, eob: (i, 0)),
            pl.BlockSpec((1, D_FF, D_MODEL), lambda i, eob: (eob[i], 0, 0)),
            pl.BlockSpec((1, 1, D_MODEL), lambda i, eob: (eob[i], 0, 0)),
        ],
        out_specs=pl.BlockSpec((BLK_M, D_MODEL), lambda i, eob: (i, 0)),
    )
    return pl.pallas_call(
        _ffn2_body,
        grid_spec=gs2,
        out_shape=jax.ShapeDtypeStruct((R_PAD, D_MODEL), jnp.float32),
    )(eob, h, w2, b2)


_C_PER_W = N_TOKENS // NW
_C_CH = 32
_C_COLS = D_MODEL // L


def _combine_body(outg_hbm, rows_hbm, w_hbm, out_hbm,
                  i0_v, i1_v, w0_v, w1_v, g0_v, g1_v, ob_v, sem0, sem1):
    wid = lax.axis_index("s") * NC + lax.axis_index("c")
    base = wid * _C_PER_W
    def body(ci, _):
        off = base + ci * _C_CH
        pltpu.sync_copy(rows_hbm.at[pl.ds(off, _C_CH)], i0_v)
        pltpu.sync_copy(rows_hbm.at[pl.ds(N_TOKENS + off, _C_CH)], i1_v)
        pltpu.sync_copy(w_hbm.at[pl.ds(off, _C_CH)],
                        w0_v.at[pl.ds(0, _C_CH)])
        pltpu.sync_copy(w_hbm.at[pl.ds(N_TOKENS + off, _C_CH)],
                        w1_v.at[pl.ds(0, _C_CH)])
        cp0 = pltpu.async_copy(outg_hbm.at[i0_v], g0_v, sem0)
        cp1 = pltpu.async_copy(outg_hbm.at[i1_v], g1_v, sem1)
        cp0.wait()
        cp1.wait()
        def tok_body(r, _):
            w0 = w0_v[pl.ds(r, L)][0]
            w1 = w1_v[pl.ds(r, L)][0]
            for c in range(_C_COLS):
                ob_v[r, pl.ds(c * L, L)] = (
                    w0 * g0_v[r, pl.ds(c * L, L)]
                    + w1 * g1_v[r, pl.ds(c * L, L)])
            return 0
        lax.fori_loop(0, _C_CH, tok_body, 0)
        pltpu.sync_copy(ob_v, out_hbm.at[pl.ds(off, _C_CH)])
        return 0
    lax.fori_loop(0, _C_PER_W // _C_CH, body, 0)


def _combine(out_g, rows, w_pairs):
    mesh = plsc.VectorSubcoreMesh(**_SC_MESH)
    f = pl.kernel(
        _combine_body,
        out_type=jax.ShapeDtypeStruct((N_TOKENS, D_MODEL), jnp.float32),
        mesh=mesh,
        scratch_types=[
            pltpu.VMEM((_C_CH,), jnp.int32),
            pltpu.VMEM((_C_CH,), jnp.int32),
            pltpu.VMEM((_C_CH + L,), jnp.float32),
            pltpu.VMEM((_C_CH + L,), jnp.float32),
            pltpu.VMEM((_C_CH, D_MODEL), jnp.float32),
            pltpu.VMEM((_C_CH, D_MODEL), jnp.float32),
            pltpu.VMEM((_C_CH, D_MODEL), jnp.float32),
            pltpu.SemaphoreType.DMA,
            pltpu.SemaphoreType.DMA,
        ],
    )
    return f(out_g, rows, w_pairs)


def kernel(x, Wg, W1, b1, W2, b2):
    B, T, D = x.shape
    x_flat = x.reshape(N_TOKENS, D)

    e2, w2p, _ps, _cs, aux = _router(x_flat, Wg)
    e_flat = e2.reshape(N_PAIRS)
    w_pairs = w2p.reshape(N_PAIRS)

    rows, eob = _dispatch(e_flat)
    xg = _dgather(x_flat, rows)
    out_g = _expert_ffn(eob, xg, W1, b1[:, None, :], W2, b2[:, None, :])
    out = _combine(out_g, rows, w_pairs).reshape(B, T, D)
    return (out, aux.reshape(()))


# speedup vs baseline: 3.9044x; 1.0559x over previous
import functools

import jax
import jax.numpy as jnp
from jax import lax
from jax.experimental import pallas as pl
from jax.experimental.pallas import tpu as pltpu
from jax.experimental.pallas import tpu_sc as plsc

D_MODEL = 1024
D_FF = 4096
N_EXPERTS = 16
TOP_K = 2
AUX_COEF = 0.01

N_TOKENS = 2 * 4096
N_PAIRS = N_TOKENS * TOP_K
BLK_M = 256
R_PAD = N_PAIRS + N_EXPERTS * BLK_M
NB = R_PAD // BLK_M

_SC_MESH = dict(core_axis_name="c", subcore_axis_name="s")
NC, NS = 2, 16
NW = NC * NS
L = 16


_RT_BLK = 512
_RT_GRID = N_TOKENS // _RT_BLK


def _router_body(x_ref, wg_ref, e_ref, w_ref,
                 ps_ref, cs_ref, aux_ref):
    i = pl.program_id(0)
    logits = jnp.dot(x_ref[...], wg_ref[...],
                     preferred_element_type=jnp.float32)
    m = jnp.max(logits, axis=1, keepdims=True)
    ex = jnp.exp(logits - m)
    probs = ex / jnp.sum(ex, axis=1, keepdims=True)

    idx16 = lax.broadcasted_iota(jnp.int32, (_RT_BLK, N_EXPERTS), 1)
    p1 = jnp.max(probs, axis=1)
    i1 = jnp.min(jnp.where(probs == p1[:, None], idx16, N_EXPERTS), axis=1)
    masked = jnp.where(idx16 == i1[:, None], -1.0, probs)
    p2 = jnp.max(masked, axis=1)
    i2 = jnp.min(jnp.where(masked == p2[:, None], idx16, N_EXPERTS), axis=1)
    denom = p1 + p2 + 1e-9
    e_ref[...] = jnp.concatenate([i1[None, :], i2[None, :]], axis=0)
    w_ref[...] = jnp.concatenate([(p1 / denom)[None, :],
                                  (p2 / denom)[None, :]], axis=0)

    @pl.when(i == 0)
    def _():
        ps_ref[...] = jnp.zeros((N_EXPERTS,), jnp.float32)
        cs_ref[...] = jnp.zeros((N_EXPERTS,), jnp.float32)

    ps_ref[...] += jnp.sum(probs, axis=0)
    cs_ref[...] += jnp.sum((idx16 == i1[:, None]).astype(jnp.float32), axis=0)

    @pl.when(i == _RT_GRID - 1)
    def _():
        f = cs_ref[...] / N_TOKENS
        P = ps_ref[...] / N_TOKENS
        aux_ref[...] = jnp.full((1, 1), AUX_COEF * N_EXPERTS * jnp.sum(f * P),
                                jnp.float32)


def _router(x_flat, Wg):
    return pl.pallas_call(
        _router_body,
        grid=(_RT_GRID,),
        in_specs=[
            pl.BlockSpec((_RT_BLK, D_MODEL), lambda i: (i, 0)),
            pl.BlockSpec((D_MODEL, N_EXPERTS), lambda i: (0, 0)),
        ],
        out_specs=[
            pl.BlockSpec((TOP_K, _RT_BLK), lambda i: (0, i)),
            pl.BlockSpec((TOP_K, _RT_BLK), lambda i: (0, i)),
            pl.BlockSpec((N_EXPERTS,), lambda i: (0,)),
            pl.BlockSpec((N_EXPERTS,), lambda i: (0,)),
            pl.BlockSpec((1, 1), lambda i: (0, 0)),
        ],
        out_shape=[
            jax.ShapeDtypeStruct((TOP_K, N_TOKENS), jnp.int32),
            jax.ShapeDtypeStruct((TOP_K, N_TOKENS), jnp.float32),
            jax.ShapeDtypeStruct((N_EXPERTS,), jnp.float32),
            jax.ShapeDtypeStruct((N_EXPERTS,), jnp.float32),
            jax.ShapeDtypeStruct((1, 1), jnp.float32),
        ],
    )(x_flat, Wg)


_DP_CHUNK = N_PAIRS // NS
_DP_SUB = 512


def _dispatch_body(e_hbm, rows_hbm, eob_hbm,
                   ids_v, cnt16_v, cnts_v, dest_v, eob_v,
                   cnt_s, base_s, bstart_s,
                   counts_sh, sem):
    cid = lax.axis_index("c")
    sid = lax.axis_index("s")

    @pl.when(cid == 0)
    def _():
        lanes = lax.iota(jnp.int32, L)
        base_off = sid * _DP_CHUNK
        pltpu.sync_copy(e_hbm.at[pl.ds(base_off, _DP_CHUNK)],
                        ids_v.at[pl.ds(0, _DP_CHUNK)])

        def zero_body(i, _):
            cnt_s[i] = 0
            return 0
        lax.fori_loop(0, N_EXPERTS, zero_body, 0)

        def hist_body(i, _):
            e = ids_v[pl.ds(i, L)][0]
            cnt_s[e] = cnt_s[e] + 1
            return 0
        lax.fori_loop(0, _DP_CHUNK, hist_body, 0)

        cvec = jnp.zeros((L,), jnp.int32)
        for e in range(N_EXPERTS):
            cvec = jnp.where(lanes == e, cnt_s[e], cvec)
        cnt16_v[...] = cvec
        pltpu.sync_copy(cnt16_v,
                        counts_sh.at[pl.ds(sid * N_EXPERTS, N_EXPERTS)])
        plsc.subcore_barrier()

        pltpu.sync_copy(counts_sh, cnts_v.at[pl.ds(0, NS * N_EXPERTS)])

        def _cnt(w, e):
            return cnts_v[pl.ds(w * N_EXPERTS + e, L)][0]

        def prior_body(w, acc):
            take = w < sid
            return tuple(
                acc[e] + jnp.where(take, _cnt(w, e), 0)
                for e in range(N_EXPERTS))
        prior = lax.fori_loop(0, NS, prior_body,
                              tuple(jnp.int32(0) for _ in range(N_EXPERTS)))

        start = jnp.int32(0)
        for e in range(N_EXPERTS):
            tot = _cnt(0, e)
            for w in range(1, NS):
                tot = tot + _cnt(w, e)
            base_s[e] = start + prior[e]
            bstart_s[e] = start // BLK_M
            start = start + ((tot + (BLK_M - 1)) // BLK_M) * BLK_M

        @pl.when(sid == 0)
        def _():
            for j in range(NB // L):
                bv = lanes + j * L
                c = jnp.full((L,), -1, jnp.int32)
                for e in range(N_EXPERTS):
                    c = c + jnp.where(bv >= bstart_s[e], 1, 0)
                eob_v[pl.ds(j * L, L)] = c
            pltpu.sync_copy(eob_v, eob_hbm)

        def rank_body(v, _):
            dvec = jnp.zeros((L,), jnp.int32)
            for j in range(L):
                e = ids_v[pl.ds(v * L + j, L)][0]
                r = base_s[e]
                base_s[e] = r + 1
                dvec = jnp.where(lanes == j, r, dvec)
            dest_v[pl.ds(v * L, L)] = dvec
            return 0
        lax.fori_loop(0, _DP_CHUNK // L, rank_body, 0)
        pltpu.sync_copy(dest_v, rows_hbm.at[pl.ds(base_off, _DP_CHUNK)])


def _dispatch(e_flat):
    mesh = plsc.VectorSubcoreMesh(**_SC_MESH)
    f = pl.kernel(
        _dispatch_body,
        out_type=[
            jax.ShapeDtypeStruct((N_PAIRS,), jnp.int32),
            jax.ShapeDtypeStruct((NB,), jnp.int32),
        ],
        mesh=mesh,
        scratch_types=[
            pltpu.VMEM((_DP_CHUNK + L,), jnp.int32),
            pltpu.VMEM((L,), jnp.int32),
            pltpu.VMEM((NS * N_EXPERTS + L,), jnp.int32),
            pltpu.VMEM((_DP_CHUNK,), jnp.int32),
            pltpu.VMEM((NB,), jnp.int32),
            pltpu.SMEM((N_EXPERTS,), jnp.int32),
            pltpu.SMEM((N_EXPERTS,), jnp.int32),
            pltpu.SMEM((N_EXPERTS,), jnp.int32),
            pltpu.VMEM_SHARED((NS * N_EXPERTS,), jnp.int32),
            pltpu.SemaphoreType.DMA,
        ],
    )
    return f(e_flat)


_GW = D_MODEL
_G_PER_W = N_PAIRS // NW
_G_CH = 32
_G_NCH = _G_PER_W // _G_CH


def _dgather_body(x_hbm, rows_hbm, xg_hbm, didx_v, tok_v, buf_a, buf_b,
                  ga_s, gb_s, sa_s, sb_s):
    wid = lax.axis_index("s") * NC + lax.axis_index("c")
    base = wid * _G_PER_W
    for j in range(_G_NCH):
        pltpu.sync_copy(rows_hbm.at[pl.ds(base + j * _G_CH, _G_CH)],
                        didx_v.at[j])
    lanes = lax.iota(jnp.int32, L)
    for v in range(_G_PER_W // L):
        pv = lanes + (base + v * L)
        tv = jnp.where(pv >= N_TOKENS, pv - N_TOKENS, pv)
        tok_v[pl.ds(v * L, L)] = tv

    bufs = (buf_a, buf_b)
    gsem = (ga_s, gb_s)
    ssem = (sa_s, sb_s)
    scp = [None, None]

    def _gather(j):
        return pltpu.async_copy(
            x_hbm.at[tok_v.at[pl.ds(j * _G_CH, _G_CH)]], bufs[j & 1],
            gsem[j & 1])

    gcp = _gather(0)
    for j in range(_G_NCH):
        par = j & 1
        gcp.wait()
        if j + 1 < _G_NCH:
            if scp[1 - par] is not None:
                scp[1 - par].wait()
            gcp = _gather(j + 1)
        scp[par] = pltpu.async_copy(bufs[par], xg_hbm.at[didx_v.at[j]],
                                    ssem[par])
    scp[0].wait()
    scp[1].wait()


def _dgather(x_i32, rows):
    mesh = plsc.VectorSubcoreMesh(**_SC_MESH)
    f = pl.kernel(
        _dgather_body,
        out_type=jax.ShapeDtypeStruct((R_PAD, _GW), jnp.float32),
        mesh=mesh,
        scratch_types=[
            pltpu.VMEM((_G_NCH, _G_CH), jnp.int32),
            pltpu.VMEM((_G_PER_W,), jnp.int32),
            pltpu.VMEM((_G_CH, _GW), jnp.float32),
            pltpu.VMEM((_G_CH, _GW), jnp.float32),
            pltpu.SemaphoreType.DMA,
            pltpu.SemaphoreType.DMA,
            pltpu.SemaphoreType.DMA,
            pltpu.SemaphoreType.DMA,
        ],
    )
    return f(x_i32, rows)


def _ffn1_body(eob_ref, xg_ref, w1_ref, b1_ref, h_ref):
    h = jnp.dot(xg_ref[...], w1_ref[0], preferred_element_type=jnp.float32)
    h = h + b1_ref[0]
    h_ref[...] = (h * jax.nn.sigmoid(h)).astype(jnp.bfloat16)


def _ffn2_body(eob_ref, h_ref, w2_ref, b2_ref, out_ref):
    o = jnp.dot(h_ref[...].astype(jnp.float32), w2_ref[0],
                preferred_element_type=jnp.float32)
    out_ref[...] = o + b2_ref[0]


def _expert_ffn(eob, xg, w1, b1, w2, b2):
    gs1 = pltpu.PrefetchScalarGridSpec(
        num_scalar_prefetch=1,
        grid=(NB,),
        in_specs=[
            pl.BlockSpec((BLK_M, D_MODEL), lambda i, eob: (i, 0)),
            pl.BlockSpec((1, D_MODEL, D_FF), lambda i, eob: (eob[i], 0, 0)),
            pl.BlockSpec((1, 1, D_FF), lambda i, eob: (eob[i], 0, 0)),
        ],
        out_specs=pl.BlockSpec((BLK_M, D_FF), lambda i, eob: (i, 0)),
    )
    h = pl.pallas_call(
        _ffn1_body,
        grid_spec=gs1,
        out_shape=jax.ShapeDtypeStruct((R_PAD, D_FF), jnp.bfloat16),
    )(eob, xg, w1, b1)
    gs2 = pltpu.PrefetchScalarGridSpec(
        num_scalar_prefetch=1,
        grid=(NB,),
        in_specs=[
            pl.BlockSpec((BLK_M, D_FF), lambda i, eob: (i, 0)),
            pl.BlockSpec((1, D_FF, D_MODEL), lambda i, eob: (eob[i], 0, 0)),
            pl.BlockSpec((1, 1, D_MODEL), lambda i, eob: (eob[i], 0, 0)),
        ],
        out_specs=pl.BlockSpec((BLK_M, D_MODEL), lambda i, eob: (i, 0)),
    )
    return pl.pallas_call(
        _ffn2_body,
        grid_spec=gs2,
        out_shape=jax.ShapeDtypeStruct((R_PAD, D_MODEL), jnp.float32),
    )(eob, h, w2, b2)


_C_PER_W = N_TOKENS // NW
_C_CH = 16
_C_NCH = _C_PER_W // _C_CH
_C_COLS = D_MODEL // L


def _combine_body(outg_hbm, rows_hbm, w_hbm, out_hbm,
                  i0_v, i1_v, w0_v, w1_v,
                  g0a_v, g1a_v, g0b_v, g1b_v, ob_v,
                  s0a, s1a, s0b, s1b):
    wid = lax.axis_index("s") * NC + lax.axis_index("c")
    base = wid * _C_PER_W
    pltpu.sync_copy(rows_hbm.at[pl.ds(base, _C_PER_W)], i0_v)
    pltpu.sync_copy(rows_hbm.at[pl.ds(N_TOKENS + base, _C_PER_W)], i1_v)
    pltpu.sync_copy(w_hbm.at[pl.ds(base, _C_PER_W)],
                    w0_v.at[pl.ds(0, _C_PER_W)])
    pltpu.sync_copy(w_hbm.at[pl.ds(N_TOKENS + base, _C_PER_W)],
                    w1_v.at[pl.ds(0, _C_PER_W)])

    g0 = (g0a_v, g0b_v)
    g1 = (g1a_v, g1b_v)
    sg0 = (s0a, s0b)
    sg1 = (s1a, s1b)

    def _fire(ci):
        par = ci & 1
        c0 = pltpu.async_copy(
            outg_hbm.at[i0_v.at[pl.ds(ci * _C_CH, _C_CH)]], g0[par],
            sg0[par])
        c1 = pltpu.async_copy(
            outg_hbm.at[i1_v.at[pl.ds(ci * _C_CH, _C_CH)]], g1[par],
            sg1[par])
        return c0, c1

    cps = _fire(0)
    for ci in range(_C_NCH):
        par = ci & 1
        cps[0].wait()
        cps[1].wait()
        if ci + 1 < _C_NCH:
            cps = _fire(ci + 1)
        def tok_body(r, _):
            w0 = w0_v[pl.ds(ci * _C_CH + r, L)][0]
            w1 = w1_v[pl.ds(ci * _C_CH + r, L)][0]
            for c in range(_C_COLS):
                ob_v[r, pl.ds(c * L, L)] = (
                    w0 * g0[par][r, pl.ds(c * L, L)]
                    + w1 * g1[par][r, pl.ds(c * L, L)])
            return 0
        lax.fori_loop(0, _C_CH, tok_body, 0)
        pltpu.sync_copy(ob_v, out_hbm.at[pl.ds(base + ci * _C_CH, _C_CH)])


def _combine(out_g, rows, w_pairs):
    mesh = plsc.VectorSubcoreMesh(**_SC_MESH)
    f = pl.kernel(
        _combine_body,
        out_type=jax.ShapeDtypeStruct((N_TOKENS, D_MODEL), jnp.float32),
        mesh=mesh,
        scratch_types=[
            pltpu.VMEM((_C_PER_W,), jnp.int32),
            pltpu.VMEM((_C_PER_W,), jnp.int32),
            pltpu.VMEM((_C_PER_W + L,), jnp.float32),
            pltpu.VMEM((_C_PER_W + L,), jnp.float32),
            pltpu.VMEM((_C_CH, D_MODEL), jnp.float32),
            pltpu.VMEM((_C_CH, D_MODEL), jnp.float32),
            pltpu.VMEM((_C_CH, D_MODEL), jnp.float32),
            pltpu.VMEM((_C_CH, D_MODEL), jnp.float32),
            pltpu.VMEM((_C_CH, D_MODEL), jnp.float32),
            pltpu.SemaphoreType.DMA,
            pltpu.SemaphoreType.DMA,
            pltpu.SemaphoreType.DMA,
            pltpu.SemaphoreType.DMA,
        ],
    )
    return f(out_g, rows, w_pairs)


def kernel(x, Wg, W1, b1, W2, b2):
    B, T, D = x.shape
    x_flat = x.reshape(N_TOKENS, D)

    e2, w2p, _ps, _cs, aux = _router(x_flat, Wg)
    e_flat = e2.reshape(N_PAIRS)
    w_pairs = w2p.reshape(N_PAIRS)

    rows, eob = _dispatch(e_flat)
    xg = _dgather(x_flat, rows)
    out_g = _expert_ffn(eob, xg, W1, b1[:, None, :], W2, b2[:, None, :])
    out = _combine(out_g, rows, w_pairs).reshape(B, T, D)
    return (out, aux.reshape(()))
